# Initial kernel scaffold; baseline (speedup 1.0000x reference)
import jax
import jax.numpy as jnp
from jax.experimental import pallas as pl


def kernel(inputs, embed, W1, b1, W2, b2, W3, b3):
    raise NotImplementedError("write your pallas kernel here")


# speedup vs baseline: 6.7951x; 6.7951x over previous
import jax
import jax.numpy as jnp
from jax import lax
from jax.experimental import pallas as pl
from jax.experimental.pallas import tpu as pltpu
from jax.experimental.pallas import tpu_sc as plsc

B = 4096
L = 50
E = 128
H = 512
O = 128

NC = 2
NS = 16
NW = NC * NS
BPW = B // NW
CHUNK_ROWS = 2
IPC = CHUNK_ROWS * L
CPW = BPW // CHUNK_ROWS
EV = E // 16


def _bag_body(idx_hbm, table_hbm, out_hbm, idx_v, gbuf, acc, sem):
    wid = lax.axis_index("s") * NC + lax.axis_index("c")
    pltpu.sync_copy(idx_hbm.at[pl.ds(wid * CPW, CPW)], idx_v)

    def chunk_body(c, carry):
        pltpu.async_copy(table_hbm.at[idx_v.at[c]], gbuf, sem).wait()
        for r in range(CHUNK_ROWS):
            def add_body(j, vecs):
                base = r * L + j
                return tuple(vecs[k] + gbuf[base, pl.ds(16 * k, 16)]
                             for k in range(EV))
            init = tuple(gbuf[r * L, pl.ds(16 * k, 16)] for k in range(EV))
            vecs = lax.fori_loop(1, L, add_body, init)
            for k in range(EV):
                acc[c * CHUNK_ROWS + r, pl.ds(16 * k, 16)] = vecs[k]
        return carry

    lax.fori_loop(0, CPW, chunk_body, 0)
    pltpu.sync_copy(acc, out_hbm.at[pl.ds(wid * BPW, BPW)])


_bag = pl.kernel(
    _bag_body,
    mesh=plsc.VectorSubcoreMesh(core_axis_name="c", subcore_axis_name="s"),
    out_type=jax.ShapeDtypeStruct((B, E), jnp.float32),
    scratch_types=[
        pltpu.VMEM((CPW, IPC), jnp.int32),
        pltpu.VMEM((IPC, E), jnp.float32),
        pltpu.VMEM((BPW, E), jnp.float32),
        pltpu.SemaphoreType.DMA,
    ],
)


BM = 512


def _mlp_body(x_ref, w1_ref, b1_ref, w2_ref, b2_ref, w3_ref, b3_ref, o_ref):
    x = x_ref[...]
    h = jnp.tanh(jnp.dot(x, w1_ref[...],
                         preferred_element_type=jnp.float32) + b1_ref[...])
    h = jnp.tanh(jnp.dot(h, w2_ref[...],
                         preferred_element_type=jnp.float32) + b2_ref[...])
    o_ref[...] = jnp.dot(h, w3_ref[...],
                         preferred_element_type=jnp.float32) + b3_ref[...]


def kernel(inputs, embed, W1, b1, W2, b2, W3, b3):
    idx2d = inputs.reshape(B // CHUNK_ROWS, IPC)
    x = _bag(idx2d, embed)
    mlp = pl.pallas_call(
        _mlp_body,
        grid=(B // BM,),
        in_specs=[
            pl.BlockSpec((BM, E), lambda i: (i, 0)),
            pl.BlockSpec((E, H), lambda i: (0, 0)),
            pl.BlockSpec((1, H), lambda i: (0, 0)),
            pl.BlockSpec((H, H), lambda i: (0, 0)),
            pl.BlockSpec((1, H), lambda i: (0, 0)),
            pl.BlockSpec((H, O), lambda i: (0, 0)),
            pl.BlockSpec((1, O), lambda i: (0, 0)),
        ],
        out_specs=pl.BlockSpec((BM, O), lambda i: (i, 0)),
        out_shape=jax.ShapeDtypeStruct((B, O), jnp.float32),
    )
    return mlp(x, W1, b1.reshape(1, H), W2, b2.reshape(1, H),
               W3, b3.reshape(1, O))


# speedup vs baseline: 10.4445x; 1.5371x over previous
import jax
import jax.numpy as jnp
from jax import lax
from jax.experimental import pallas as pl
from jax.experimental.pallas import tpu as pltpu
from jax.experimental.pallas import tpu_sc as plsc

B = 4096
L = 50
E = 128
H = 512
O = 128

NC = 2
NS = 16
NW = NC * NS
BPW = B // NW
CHUNK_ROWS = 2
IPC = CHUNK_ROWS * L
CPW = BPW // CHUNK_ROWS
EV = E // 16


UNROLL = 5


def _bag_body(idx_hbm, table_hbm, out_hbm, idx_v, gbuf0, gbuf1, acc,
              sem0, sem1):
    wid = lax.axis_index("s") * NC + lax.axis_index("c")
    gbufs = (gbuf0, gbuf1)
    sems = (sem0, sem1)
    pltpu.sync_copy(idx_hbm.at[pl.ds(wid * CPW, CPW)], idx_v)

    def start(c, b):
        pltpu.make_async_copy(table_hbm.at[idx_v.at[c]], gbufs[b],
                              sems[b]).start()

    def wait(b):
        pltpu.make_async_copy(table_hbm.at[idx_v.at[0]], gbufs[b],
                              sems[b]).wait()

    def accum(b, c):
        gb = gbufs[b]
        for r in range(CHUNK_ROWS):
            row0 = r * L

            def add_body(jo, vecs):
                base = row0 + jo * UNROLL
                for u in range(UNROLL):
                    vecs = tuple(vecs[k] + gb[base + u, pl.ds(16 * k, 16)]
                                 for k in range(EV))
                return vecs

            init = tuple(jnp.zeros((16,), jnp.float32) for _ in range(EV))
            vecs = lax.fori_loop(0, L // UNROLL, add_body, init)
            for k in range(EV):
                acc[c * CHUNK_ROWS + r, pl.ds(16 * k, 16)] = vecs[k]

    start(0, 0)
    start(1, 1)

    def loop_body(i, carry):
        c0 = 2 * i
        wait(0)
        accum(0, c0)
        start(c0 + 2, 0)
        wait(1)
        accum(1, c0 + 1)
        start(c0 + 3, 1)
        return carry

    lax.fori_loop(0, (CPW - 2) // 2, loop_body, 0)
    wait(0)
    accum(0, CPW - 2)
    wait(1)
    accum(1, CPW - 1)
    pltpu.sync_copy(acc, out_hbm.at[pl.ds(wid * BPW, BPW)])


_bag = pl.kernel(
    _bag_body,
    mesh=plsc.VectorSubcoreMesh(core_axis_name="c", subcore_axis_name="s"),
    out_type=jax.ShapeDtypeStruct((B, E), jnp.float32),
    scratch_types=[
        pltpu.VMEM((CPW, IPC), jnp.int32),
        pltpu.VMEM((IPC, E), jnp.float32),
        pltpu.VMEM((IPC, E), jnp.float32),
        pltpu.VMEM((BPW, E), jnp.float32),
        pltpu.SemaphoreType.DMA,
        pltpu.SemaphoreType.DMA,
    ],
)


BM = 512


def _mlp_body(x_ref, w1_ref, b1_ref, w2_ref, b2_ref, w3_ref, b3_ref, o_ref):
    x = x_ref[...]
    h = jnp.tanh(jnp.dot(x, w1_ref[...],
                         preferred_element_type=jnp.float32) + b1_ref[...])
    h = jnp.tanh(jnp.dot(h, w2_ref[...],
                         preferred_element_type=jnp.float32) + b2_ref[...])
    o_ref[...] = jnp.dot(h, w3_ref[...],
                         preferred_element_type=jnp.float32) + b3_ref[...]


def kernel(inputs, embed, W1, b1, W2, b2, W3, b3):
    idx2d = inputs.reshape(B // CHUNK_ROWS, IPC)
    x = _bag(idx2d, embed)
    mlp = pl.pallas_call(
        _mlp_body,
        grid=(B // BM,),
        in_specs=[
            pl.BlockSpec((BM, E), lambda i: (i, 0)),
            pl.BlockSpec((E, H), lambda i: (0, 0)),
            pl.BlockSpec((1, H), lambda i: (0, 0)),
            pl.BlockSpec((H, H), lambda i: (0, 0)),
            pl.BlockSpec((1, H), lambda i: (0, 0)),
            pl.BlockSpec((H, O), lambda i: (0, 0)),
            pl.BlockSpec((1, O), lambda i: (0, 0)),
        ],
        out_specs=pl.BlockSpec((BM, O), lambda i: (i, 0)),
        out_shape=jax.ShapeDtypeStruct((B, O), jnp.float32),
    )
    return mlp(x, W1, b1.reshape(1, H), W2, b2.reshape(1, H),
               W3, b3.reshape(1, O))
